# baseline (speedup 1.0000x reference)
import functools

import jax
import jax.numpy as jnp
from jax import lax
from jax.experimental import pallas as pl
from jax.experimental.pallas import tpu as pltpu
from jax.experimental.pallas import tpu_sc as plsc

N = 10000
C = 128
HEADS = 6
E = 320000
TRASH = N
NPAD = 10240
CHUNK = 128
CPT = 80
EROWS = 32 * CPT
RROWS = E // CHUNK
ROWS_PER_TILE = NPAD // 16

_mesh = plsc.VectorSubcoreMesh(core_axis_name="c", subcore_axis_name="s")


def _adj_body(ei_ref, row_ref, adj_ref):
    r0 = ei_ref[0]
    c0 = ei_ref[1]
    adj_real = jnp.where(r0 == c0, TRASH, c0)
    pad = EROWS - RROWS
    row_ref[...] = jnp.concatenate(
        [r0, jnp.zeros((pad, CHUNK), jnp.int32)], axis=0)
    adj_ref[...] = jnp.concatenate(
        [adj_real, jnp.full((pad, CHUNK), TRASH, jnp.int32)], axis=0)


def _adj_call(ei3):
    return pl.pallas_call(
        _adj_body,
        out_shape=(jax.ShapeDtypeStruct((EROWS, CHUNK), jnp.int32),
                   jax.ShapeDtypeStruct((EROWS, CHUNK), jnp.int32)),
    )(ei3)


def _deg_body(adj_hbm, deg_out, adji_v, ones_v, zrow_v, acc, sem):
    cid = lax.axis_index("c")
    sid = lax.axis_index("s")
    wid = cid * 16 + sid

    ones16 = jnp.ones((16,), jnp.float32)
    zeros16 = jnp.zeros((16,), jnp.float32)
    for j in range(CHUNK // 16):
        ones_v[pl.ds(j * 16, 16)] = ones16
    for j in range(ROWS_PER_TILE // 16):
        zrow_v[pl.ds(j * 16, 16)] = zeros16
    pltpu.sync_copy(adj_hbm.at[pl.ds(wid * CPT, CPT)], adji_v)
    pltpu.sync_copy(zrow_v, acc.at[pl.ds(sid * ROWS_PER_TILE, ROWS_PER_TILE)])
    plsc.subcore_barrier()

    def group_body(gi, carry):
        for b in range(10):
            i = gi * 10 + b
            pltpu.async_copy(ones_v, acc.at[adji_v.at[i]], sem, add=True)
        for b in range(10):
            i = gi * 10 + b
            pltpu.make_async_copy(ones_v, acc.at[adji_v.at[i]], sem).wait()
        return carry

    lax.fori_loop(0, CPT // 10, group_body, 0)
    plsc.subcore_barrier()
    pltpu.sync_copy(acc.at[pl.ds(sid * ROWS_PER_TILE, ROWS_PER_TILE)],
                    deg_out.at[cid, pl.ds(sid * ROWS_PER_TILE, ROWS_PER_TILE)])


_deg_call = functools.partial(
    pl.kernel,
    out_type=jax.ShapeDtypeStruct((2, NPAD), jnp.float32),
    mesh=_mesh,
    scratch_types=[
        pltpu.VMEM((CPT, CHUNK), jnp.int32),
        pltpu.VMEM((CHUNK,), jnp.float32),
        pltpu.VMEM((ROWS_PER_TILE,), jnp.float32),
        pltpu.VMEM_SHARED((NPAD,), jnp.float32),
        pltpu.SemaphoreType.DMA,
    ],
)(_deg_body)


_GSZ = 16


def _agg_body(g_hbm, row_hbm, adj_hbm, out_hbm,
              rowi_v, adji_v, b0, b1, acc, s0, s1):
    cid = lax.axis_index("c")
    sid = lax.axis_index("s")
    wid = cid * 16 + sid

    zeros16 = jnp.zeros((16,), jnp.float32)

    def zrow(i, carry):
        for j in range(C // 16):
            b0[i, pl.ds(j * 16, 16)] = zeros16
        return carry

    lax.fori_loop(0, CHUNK, zrow, 0)

    def zacc(k, carry):
        pltpu.sync_copy(
            b0, acc.at[pl.ds(sid * ROWS_PER_TILE + k * CHUNK, CHUNK)])
        return carry

    lax.fori_loop(0, ROWS_PER_TILE // CHUNK, zacc, 0)
    plsc.subcore_barrier()

    bufs = (b0, b1)
    sems = (s0, s1)

    def group_body(g, carry):
        base = wid * CPT + g * _GSZ
        pltpu.sync_copy(row_hbm.at[pl.ds(base, _GSZ)], rowi_v)
        pltpu.sync_copy(adj_hbm.at[pl.ds(base, _GSZ)], adji_v)
        for b in range(2):
            pltpu.async_copy(g_hbm.at[rowi_v.at[b]], bufs[b], sems[b])

        def pair_body(k, c2):
            for b in range(2):
                i = k * 2 + b
                pltpu.make_async_copy(g_hbm.at[rowi_v.at[i]], bufs[b],
                                      sems[b]).wait()
                pltpu.sync_copy(bufs[b], acc.at[adji_v.at[i]], add=True)
                ip = jnp.minimum(i + 2, _GSZ - 1)
                pltpu.async_copy(g_hbm.at[rowi_v.at[ip]], bufs[b], sems[b])
            return c2

        lax.fori_loop(0, _GSZ // 2, pair_body, 0)
        for b in range(2):
            pltpu.make_async_copy(g_hbm.at[rowi_v.at[0]], bufs[b],
                                  sems[b]).wait()
        return carry

    lax.fori_loop(0, CPT // _GSZ, group_body, 0)
    plsc.subcore_barrier()
    pltpu.sync_copy(acc.at[pl.ds(sid * ROWS_PER_TILE, ROWS_PER_TILE)],
                    out_hbm.at[cid, pl.ds(sid * ROWS_PER_TILE, ROWS_PER_TILE)])


_agg_call = functools.partial(
    pl.kernel,
    out_type=jax.ShapeDtypeStruct((2, NPAD, C), jnp.float32),
    mesh=_mesh,
    scratch_types=[
        pltpu.VMEM((_GSZ, CHUNK), jnp.int32),
        pltpu.VMEM((_GSZ, CHUNK), jnp.int32),
        pltpu.VMEM((CHUNK, C), jnp.float32),
        pltpu.VMEM((CHUNK, C), jnp.float32),
        pltpu.VMEM_SHARED((NPAD, C), jnp.float32),
        pltpu.SemaphoreType.DMA,
        pltpu.SemaphoreType.DMA,
    ],
)(_agg_body)


_RB = 640


def _g_body(deg_ref, x_ref, w_ref, g_ref):
    p = deg_ref[...]
    s = lax.rsqrt(p[0] + p[1] + 1.0)
    h = jnp.dot(x_ref[...], w_ref[...], preferred_element_type=jnp.float32)
    g_ref[...] = s[:, None] * h


def _g_call(deg_parts, x, W_gcn):
    return pl.pallas_call(
        _g_body,
        grid=(NPAD // _RB,),
        in_specs=[
            pl.BlockSpec((2, _RB), lambda i: (0, i)),
            pl.BlockSpec((_RB, C), lambda i: (i, 0)),
            pl.BlockSpec((C, C), lambda i: (0, 0)),
        ],
        out_specs=pl.BlockSpec((_RB, C), lambda i: (i, 0)),
        out_shape=jax.ShapeDtypeStruct((NPAD, C), jnp.float32),
    )(deg_parts, x, W_gcn)


def _out_body(deg_ref, a_ref, g_ref, b_ref, w_ref, o_ref):
    p = deg_ref[...]
    a = a_ref[...]
    s = lax.rsqrt(p[0] + p[1] + 1.0)
    out1 = s[:, None] * (a[0] + a[1] + g_ref[...]) + b_ref[...]
    o_ref[...] = jnp.dot(out1, w_ref[...], preferred_element_type=jnp.float32)


def _out_call(deg_parts, a_parts, g, b2d, weight):
    return pl.pallas_call(
        _out_body,
        grid=(NPAD // _RB,),
        in_specs=[
            pl.BlockSpec((2, _RB), lambda i: (0, i)),
            pl.BlockSpec((2, _RB, C), lambda i: (0, i, 0)),
            pl.BlockSpec((_RB, C), lambda i: (i, 0)),
            pl.BlockSpec((1, C), lambda i: (0, 0)),
            pl.BlockSpec((C, HEADS * C), lambda i: (0, 0)),
        ],
        out_specs=pl.BlockSpec((_RB, HEADS * C), lambda i: (i, 0)),
        out_shape=jax.ShapeDtypeStruct((N, HEADS * C), jnp.float32),
    )(deg_parts, a_parts, g, b2d, weight)


@jax.jit
def kernel(x, edge_index, old_index, W_gcn, b_gcn, weight):
    ei3 = edge_index.reshape(2, RROWS, CHUNK)
    row2d, adj2d = _adj_call(ei3)
    deg_parts = _deg_call(adj2d)
    g = _g_call(deg_parts, x, W_gcn)
    a_parts = _agg_call(g, row2d, adj2d)
    out = _out_call(deg_parts, a_parts, g, b_gcn.reshape(1, C), weight)
    return out.reshape(N, HEADS, C)


# speedup vs baseline: 1.0023x; 1.0023x over previous
import functools

import jax
import jax.numpy as jnp
from jax import lax
from jax.experimental import pallas as pl
from jax.experimental.pallas import tpu as pltpu
from jax.experimental.pallas import tpu_sc as plsc

N = 10000
C = 128
HEADS = 6
E = 320000
TRASH = N
NPAD = 10240
CHUNK = 128
CPT = 80
EROWS = 32 * CPT
RROWS = E // CHUNK
ROWS_PER_TILE = NPAD // 16

_mesh = plsc.VectorSubcoreMesh(core_axis_name="c", subcore_axis_name="s")


def _adj_body(ei_ref, row_ref, adj_ref):
    r0 = ei_ref[0]
    c0 = ei_ref[1]
    adj_real = jnp.where(r0 == c0, TRASH, c0)
    pad = EROWS - RROWS
    row_ref[...] = jnp.concatenate(
        [r0, jnp.zeros((pad, CHUNK), jnp.int32)], axis=0)
    adj_ref[...] = jnp.concatenate(
        [adj_real, jnp.full((pad, CHUNK), TRASH, jnp.int32)], axis=0)


def _adj_call(ei3):
    return pl.pallas_call(
        _adj_body,
        out_shape=(jax.ShapeDtypeStruct((EROWS, CHUNK), jnp.int32),
                   jax.ShapeDtypeStruct((EROWS, CHUNK), jnp.int32)),
    )(ei3)


def _deg_body(adj_hbm, deg_out, adji_v, ones_v, zrow_v, acc, sem):
    cid = lax.axis_index("c")
    sid = lax.axis_index("s")
    wid = cid * 16 + sid

    ones16 = jnp.ones((16,), jnp.float32)
    zeros16 = jnp.zeros((16,), jnp.float32)
    for j in range(CHUNK // 16):
        ones_v[pl.ds(j * 16, 16)] = ones16
    for j in range(ROWS_PER_TILE // 16):
        zrow_v[pl.ds(j * 16, 16)] = zeros16
    pltpu.sync_copy(adj_hbm.at[pl.ds(wid * CPT, CPT)], adji_v)
    pltpu.sync_copy(zrow_v, acc.at[pl.ds(sid * ROWS_PER_TILE, ROWS_PER_TILE)])
    plsc.subcore_barrier()

    def group_body(gi, carry):
        for b in range(10):
            i = gi * 10 + b
            pltpu.async_copy(ones_v, acc.at[adji_v.at[i]], sem, add=True)
        for b in range(10):
            i = gi * 10 + b
            pltpu.make_async_copy(ones_v, acc.at[adji_v.at[i]], sem).wait()
        return carry

    lax.fori_loop(0, CPT // 10, group_body, 0)
    plsc.subcore_barrier()
    pltpu.sync_copy(acc.at[pl.ds(sid * ROWS_PER_TILE, ROWS_PER_TILE)],
                    deg_out.at[cid, pl.ds(sid * ROWS_PER_TILE, ROWS_PER_TILE)])


_deg_call = functools.partial(
    pl.kernel,
    out_type=jax.ShapeDtypeStruct((2, NPAD), jnp.float32),
    mesh=_mesh,
    scratch_types=[
        pltpu.VMEM((CPT, CHUNK), jnp.int32),
        pltpu.VMEM((CHUNK,), jnp.float32),
        pltpu.VMEM((ROWS_PER_TILE,), jnp.float32),
        pltpu.VMEM_SHARED((NPAD,), jnp.float32),
        pltpu.SemaphoreType.DMA,
    ],
)(_deg_body)


_GSZ = 16


def _agg_body(g_hbm, row_hbm, adj_hbm, out_hbm,
              rowi_v, adji_v, b0, b1, acc, s0, s1):
    cid = lax.axis_index("c")
    sid = lax.axis_index("s")
    wid = cid * 16 + sid

    zeros16 = jnp.zeros((16,), jnp.float32)

    def zrow(i, carry):
        for j in range(C // 16):
            b0[i, pl.ds(j * 16, 16)] = zeros16
        return carry

    lax.fori_loop(0, CHUNK, zrow, 0)

    def zacc(k, carry):
        pltpu.sync_copy(
            b0, acc.at[pl.ds(sid * ROWS_PER_TILE + k * CHUNK, CHUNK)])
        return carry

    lax.fori_loop(0, ROWS_PER_TILE // CHUNK, zacc, 0)
    plsc.subcore_barrier()

    bufs = (b0, b1)
    sems = (s0, s1)

    def group_body(g, carry):
        base = wid * CPT + g * _GSZ
        pltpu.sync_copy(row_hbm.at[pl.ds(base, _GSZ)], rowi_v)
        pltpu.sync_copy(adj_hbm.at[pl.ds(base, _GSZ)], adji_v)
        def pair_body(k, c2):
            for b in range(2):
                i = k * 2 + b
                pltpu.async_copy(g_hbm.at[rowi_v.at[i]], bufs[b],
                                 sems[b]).wait()
                pltpu.sync_copy(bufs[b], acc.at[adji_v.at[i]], add=True)
            return c2

        lax.fori_loop(0, _GSZ // 2, pair_body, 0)
        return carry

    lax.fori_loop(0, CPT // _GSZ, group_body, 0)
    plsc.subcore_barrier()
    pltpu.sync_copy(acc.at[pl.ds(sid * ROWS_PER_TILE, ROWS_PER_TILE)],
                    out_hbm.at[cid, pl.ds(sid * ROWS_PER_TILE, ROWS_PER_TILE)])


_agg_call = functools.partial(
    pl.kernel,
    out_type=jax.ShapeDtypeStruct((2, NPAD, C), jnp.float32),
    mesh=_mesh,
    scratch_types=[
        pltpu.VMEM((_GSZ, CHUNK), jnp.int32),
        pltpu.VMEM((_GSZ, CHUNK), jnp.int32),
        pltpu.VMEM((CHUNK, C), jnp.float32),
        pltpu.VMEM((CHUNK, C), jnp.float32),
        pltpu.VMEM_SHARED((NPAD, C), jnp.float32),
        pltpu.SemaphoreType.DMA,
        pltpu.SemaphoreType.DMA,
    ],
)(_agg_body)


_RB = 640


def _g_body(deg_ref, x_ref, w_ref, g_ref):
    p = deg_ref[...]
    s = lax.rsqrt(p[0] + p[1] + 1.0)
    h = jnp.dot(x_ref[...], w_ref[...], preferred_element_type=jnp.float32)
    g_ref[...] = s[:, None] * h


def _g_call(deg_parts, x, W_gcn):
    return pl.pallas_call(
        _g_body,
        grid=(NPAD // _RB,),
        in_specs=[
            pl.BlockSpec((2, _RB), lambda i: (0, i)),
            pl.BlockSpec((_RB, C), lambda i: (i, 0)),
            pl.BlockSpec((C, C), lambda i: (0, 0)),
        ],
        out_specs=pl.BlockSpec((_RB, C), lambda i: (i, 0)),
        out_shape=jax.ShapeDtypeStruct((NPAD, C), jnp.float32),
    )(deg_parts, x, W_gcn)


def _out_body(deg_ref, a_ref, g_ref, b_ref, w_ref, o_ref):
    p = deg_ref[...]
    a = a_ref[...]
    s = lax.rsqrt(p[0] + p[1] + 1.0)
    out1 = s[:, None] * (a[0] + a[1] + g_ref[...]) + b_ref[...]
    o_ref[...] = jnp.dot(out1, w_ref[...], preferred_element_type=jnp.float32)


def _out_call(deg_parts, a_parts, g, b2d, weight):
    return pl.pallas_call(
        _out_body,
        grid=(NPAD // _RB,),
        in_specs=[
            pl.BlockSpec((2, _RB), lambda i: (0, i)),
            pl.BlockSpec((2, _RB, C), lambda i: (0, i, 0)),
            pl.BlockSpec((_RB, C), lambda i: (i, 0)),
            pl.BlockSpec((1, C), lambda i: (0, 0)),
            pl.BlockSpec((C, HEADS * C), lambda i: (0, 0)),
        ],
        out_specs=pl.BlockSpec((_RB, HEADS * C), lambda i: (i, 0)),
        out_shape=jax.ShapeDtypeStruct((N, HEADS * C), jnp.float32),
    )(deg_parts, a_parts, g, b2d, weight)


@jax.jit
def kernel(x, edge_index, old_index, W_gcn, b_gcn, weight):
    ei3 = edge_index.reshape(2, RROWS, CHUNK)
    row2d, adj2d = _adj_call(ei3)
    deg_parts = _deg_call(adj2d)
    g = _g_call(deg_parts, x, W_gcn)
    a_parts = _agg_call(g, row2d, adj2d)
    out = _out_call(deg_parts, a_parts, g, b_gcn.reshape(1, C), weight)
    return out.reshape(N, HEADS, C)
